# baseline (speedup 1.0000x reference)
import jax
import jax.numpy as jnp
from jax import lax
from jax.experimental import pallas as pl
from jax.experimental.pallas import tpu as pltpu
from jax.experimental.pallas import tpu_sc as plsc

_B, _C, _H, _W = 8, 255, 76, 76
_S = _H * _W
_NC, _NS = 2, 16
_COLS = 2 * _W
_NT = _S // _COLS
_ZROW = _COLS * 16
_NZ = _B * 16 * _NT
_PW = (10.0 / 608.0, 16.0 / 608.0, 33.0 / 608.0)
_PH = (13.0 / 608.0, 30.0 / 608.0, 23.0 / 608.0)


def _slab_consts(slab):
    is_exp, scale, inv, d0, d1 = [], [], [], [], []
    for lane in range(16):
        c = min(slab * 16 + lane, _C - 1)
        a, d = c // 85, c % 85
        is_exp.append(d in (2, 3))
        scale.append(_PW[a] if d == 2 else (_PH[a] if d == 3 else 0.0))
        inv.append(1.0 / _W if d in (0, 1) else (0.0 if d in (2, 3) else 1.0))
        d0.append(1.0 if d == 0 else 0.0)
        d1.append(1.0 if d == 1 else 0.0)
    return is_exp, scale, inv, d0, d1


def _lane_vec(vals, iota):
    out = None
    uniq = sorted(set(vals))
    base = jnp.full((16,), jnp.float32(uniq[0]))
    out = base
    for u in uniq[1:]:
        mask = jnp.zeros((16,), jnp.bool_)
        for lane, v in enumerate(vals):
            if v == u:
                mask = mask | (iota == lane)
        out = jnp.where(mask, jnp.float32(u), out)
    return out


def _decode_body(x_ref, y_ref, z_ref, sem):
    core = lax.axis_index("c")
    sid = lax.axis_index("s")
    iota = lax.iota(jnp.int32, 16)
    viota16 = iota * 16

    def pass1(inrow, rbuf):
        for uu in range(4):
            bg = core * 4 + uu
            for r in range(16):
                c = jnp.minimum(sid * 16 + r, _C - 1)
                pltpu.sync_copy(x_ref.at[bg, c, :], inrow)

                @plsc.parallel_loop(0, _S // 16, 1, unroll=2)
                def rk(k):
                    v = inrow[pl.ds(16 * k, 16)]
                    plsc.store_scatter(rbuf, [viota16 + (256 * k + r)], v)

            row0 = (bg * 16 + sid) * _NT
            descs = [
                pltpu.async_copy(
                    rbuf.at[pl.ds(row * _ZROW, _ZROW)],
                    z_ref.at[row0 + row, :],
                    sem,
                )
                for row in range(_NT)
            ]
            for d in descs:
                d.wait()

    pl.run_scoped(
        pass1,
        pltpu.VMEM((_S,), jnp.float32),
        pltpu.VMEM((_S * 16,), jnp.float32),
    )

    plsc.subcore_barrier()

    def pass2(buf2, outb):
        def do_tile(bg, j):
            idx = (bg * 16 + iota) * _NT + j
            pltpu.async_copy(z_ref.at[idx], buf2, sem).wait()

            def col(s, carry):
                wf = jnp.where(s < _W, s, s - _W).astype(jnp.float32)
                hf = (2 * j + jnp.where(s < _W, 0, 1)).astype(jnp.float32)
                vs = [buf2[slab, pl.ds(s * 16, 16)] for slab in range(16)]
                res = []
                for slab in range(16):
                    v = vs[slab]
                    is_exp, scale, inv, d0, d1 = _slab_consts(slab)
                    sig = 1.0 / (1.0 + jnp.exp(-v))
                    if not any(is_exp):
                        res.append(sig)
                        continue
                    e = jnp.exp(v)
                    mexp = _lane_vec([1.0 if t else 0.0 for t in is_exp],
                                     iota) > 0.5
                    addv = _lane_vec(d0, iota) * wf + _lane_vec(d1, iota) * hf
                    r = jnp.where(mexp, _lane_vec(scale, iota) * e,
                                  (sig + addv) * _lane_vec(inv, iota))
                    res.append(r)
                for slab in range(16):
                    outb[pl.ds(s * _C + slab * 16, 16)] = res[slab]
                return carry

            lax.fori_loop(0, _COLS, col, 0)
            pltpu.sync_copy(
                outb.at[pl.ds(0, _COLS * _C)],
                y_ref.at[bg, pl.ds(j * _COLS * _C, _COLS * _C)],
            )

        trip = jnp.where(sid < 6, 3, 2)
        for b_local in range(4):
            bg = core * 4 + b_local

            def jt(jj, carry):
                do_tile(bg, sid + 16 * jj)
                return carry

            lax.fori_loop(0, trip, jt, 0)

    pl.run_scoped(
        pass2,
        pltpu.VMEM((16, _ZROW), jnp.float32),
        pltpu.VMEM((_COLS * _C + 16,), jnp.float32),
    )


def kernel(x):
    xr = x.reshape(_B, _C, _S)
    mesh = plsc.VectorSubcoreMesh(core_axis_name="c", subcore_axis_name="s")
    y, _ = pl.kernel(
        _decode_body,
        out_type=(
            jax.ShapeDtypeStruct((_B, _S * _C), jnp.float32),
            jax.ShapeDtypeStruct((_NZ, _ZROW), jnp.float32),
        ),
        mesh=mesh,
        scratch_types=[pltpu.SemaphoreType.DMA],
        compiler_params=pltpu.CompilerParams(
            use_tc_tiling_on_sc=False, needs_layout_passes=False),
    )(xr)
    return y.reshape(_B, _S * _C // 85, 85)
